# Initial kernel scaffold; baseline (speedup 1.0000x reference)
import jax
import jax.numpy as jnp
from jax.experimental import pallas as pl


def kernel(adj_indices, adj_values, uEmbeds, iEmbeds):
    raise NotImplementedError("write your pallas kernel here")


# speedup vs baseline: 2.1811x; 2.1811x over previous
import functools

import jax
import jax.numpy as jnp
from jax import lax
from jax.experimental import pallas as pl
from jax.experimental.pallas import tpu as pltpu
from jax.experimental.pallas import tpu_sc as plsc

N_USER = 5000
N_ITEM = 5000
N = N_USER + N_ITEM
D = 128
DH = D // 2
NC = 2
NS = 16
L = 16
NPAD = 10240
ROWS_PER_TILE = NPAD // NS
K = 128
NROWCHUNKS = ROWS_PER_TILE // K
E = 320000
EPT = E // NS
NCHUNKS = -(-EPT // K)
EPT_PAD = NCHUNKS * K
EPAD = NS * EPT_PAD


def _body(x0, cols2, rows_, vals_, out, e1, accA, accB,
          colbuf, rowbuf, valbuf, gbuf, b2, sem):
    c = lax.axis_index("c")
    s = lax.axis_index("s")
    rbase = s * ROWS_PER_TILE
    ebase = s * EPT_PAD
    hbase = c * NPAD

    def zrow(r, _):
        for j in range(DH // L):
            b2[r, pl.ds(j * L, L)] = jnp.zeros((L,), jnp.float32)
        return 0
    lax.fori_loop(0, K, zrow, 0)

    def zchunk(i, _):
        pltpu.sync_copy(b2, accB.at[pl.ds(rbase + i * K, K)])
        return 0
    lax.fori_loop(0, NROWCHUNKS, zchunk, 0)
    plsc.subcore_barrier()

    def layer(src_hbm, acc):
        def chunk(t, _):
            off = ebase + t * K
            pltpu.sync_copy(cols2.at[c, pl.ds(off, K)], colbuf)
            pltpu.sync_copy(rows_.at[pl.ds(off, K)], rowbuf)
            pltpu.sync_copy(vals_.at[pl.ds(off, K)], valbuf)
            pltpu.async_copy(src_hbm.at[colbuf], gbuf, sem).wait()

            def group(g, _):
                val16 = valbuf[pl.ds(g * L, L)]
                for el in range(L):
                    e = g * L + el
                    v = val16[el]
                    for j in range(DH // L):
                        sl = pl.ds(j * L, L)
                        gbuf[e, sl] = gbuf[e, sl] * v
                return 0
            lax.fori_loop(0, K // L, group, 0)
            pltpu.sync_copy(gbuf, acc.at[rowbuf], add=True)
            return 0
        lax.fori_loop(0, NCHUNKS, chunk, 0)

    layer(x0, accB)
    plsc.subcore_barrier()

    def mid(i, _):
        r0 = rbase + i * K
        pltpu.sync_copy(accB.at[pl.ds(r0, K)], gbuf)
        pltpu.sync_copy(gbuf, e1.at[pl.ds(hbase + r0, K)])
        pltpu.sync_copy(x0.at[pl.ds(hbase + r0, K)], b2)

        def addrow(r, _):
            for j in range(DH // L):
                sl = pl.ds(j * L, L)
                b2[r, sl] = b2[r, sl] + gbuf[r, sl]
            return 0
        lax.fori_loop(0, K, addrow, 0)
        pltpu.sync_copy(b2, accA.at[pl.ds(r0, K)])
        return 0
    lax.fori_loop(0, NROWCHUNKS, mid, 0)
    plsc.subcore_barrier()

    layer(e1, accA)
    plsc.subcore_barrier()

    def fin(i, _):
        r0 = rbase + i * K
        pltpu.sync_copy(accA.at[pl.ds(r0, K)], gbuf)
        pltpu.sync_copy(gbuf, out.at[pl.ds(hbase + r0, K)])
        return 0
    lax.fori_loop(0, NROWCHUNKS, fin, 0)


@functools.partial(jax.jit, static_argnums=())
def _run(x0, cols2, rows_, vals_):
    mesh = plsc.VectorSubcoreMesh(
        core_axis_name="c", subcore_axis_name="s",
        num_cores=NC, num_subcores=NS)
    kfn = pl.kernel(
        _body,
        out_type=[
            jax.ShapeDtypeStruct((NC * NPAD, DH), jnp.float32),
            jax.ShapeDtypeStruct((NC * NPAD, DH), jnp.float32),
        ],
        mesh=mesh,
        compiler_params=pltpu.CompilerParams(use_tc_tiling_on_sc=False),
        scratch_types=[
            pltpu.VMEM_SHARED((NPAD, DH), jnp.float32),
            pltpu.VMEM_SHARED((NPAD, DH), jnp.float32),
            pltpu.VMEM((K,), jnp.int32),
            pltpu.VMEM((K,), jnp.int32),
            pltpu.VMEM((K,), jnp.float32),
            pltpu.VMEM((K, DH), jnp.float32),
            pltpu.VMEM((K, DH), jnp.float32),
            pltpu.SemaphoreType.DMA,
        ],
    )
    return kfn(x0, cols2, rows_, vals_)


def kernel(adj_indices, adj_values, uEmbeds, iEmbeds):
    emb = jnp.concatenate([uEmbeds, iEmbeds], axis=0)
    embp = jnp.zeros((NPAD, D), jnp.float32).at[:N].set(emb)
    x0 = jnp.concatenate([embp[:, :DH], embp[:, DH:]], axis=0)

    rows = adj_indices[0].astype(jnp.int32)
    cols = adj_indices[1].astype(jnp.int32)
    vals = adj_values.astype(jnp.float32)
    pad = ((0, 0), (0, EPT_PAD - EPT))
    rows_p = jnp.pad(rows.reshape(NS, EPT), pad).reshape(EPAD)
    cols_p = jnp.pad(cols.reshape(NS, EPT), pad).reshape(EPAD)
    vals_p = jnp.pad(vals.reshape(NS, EPT), pad).reshape(EPAD)
    cols2 = jnp.stack([cols_p, cols_p + NPAD])

    out, _e1 = _run(x0, cols2, rows_p, vals_p)
    total = jnp.concatenate([out[:N], out[NPAD:NPAD + N]], axis=1)
    return total[:N_USER], total[N_USER:]


# speedup vs baseline: 2.9905x; 1.3711x over previous
import functools

import jax
import jax.numpy as jnp
from jax import lax
from jax.experimental import pallas as pl
from jax.experimental.pallas import tpu as pltpu
from jax.experimental.pallas import tpu_sc as plsc

N_USER = 5000
N_ITEM = 5000
N = N_USER + N_ITEM
D = 128
DH = D // 2
NC = 2
NS = 16
L = 16
NPAD = 10240
ROWS_PER_TILE = NPAD // NS
K = 128
NROWCHUNKS = ROWS_PER_TILE // K
E = 320000
EPT = E // NS
NCHUNKS = 158
EPT_PAD = NCHUNKS * K
NGCHUNK = NS * NCHUNKS + 1


def _body(x0, edg, out, e1, accA, accB,
          ib0, ib1, gb0, gb1, b2, sem0, sem1):
    c = lax.axis_index("c")
    s = lax.axis_index("s")
    rbase = s * ROWS_PER_TILE
    gbase = s * NCHUNKS
    hbase = c * NPAD

    def zrow(r, _):
        for j in range(DH // L):
            b2[r, pl.ds(j * L, L)] = jnp.zeros((L,), jnp.float32)
        return 0
    lax.fori_loop(0, K, zrow, 0)

    def zchunk(i, _):
        pltpu.sync_copy(b2, accB.at[pl.ds(rbase + i * K, K)])
        return 0
    lax.fori_loop(0, NROWCHUNKS, zchunk, 0)
    plsc.subcore_barrier()

    def layer(src_hbm, acc):
        def step(t, ibA, gbA, semA, ibB, gbB, semB):
            pltpu.sync_copy(edg.at[c, gbase + t + 1], ibB)
            pltpu.async_copy(src_hbm.at[ibB.at[0]], gbB, semB)
            pltpu.make_async_copy(src_hbm.at[ibA.at[0]], gbA, semA).wait()

            def group(g, _):
                val16 = plsc.bitcast(ibA[2, pl.ds(g * L, L)], jnp.float32)
                for el in range(L):
                    e = g * L + el
                    v = val16[el]
                    for j in range(DH // L):
                        sl = pl.ds(j * L, L)
                        gbA[e, sl] = gbA[e, sl] * v
                return 0
            lax.fori_loop(0, K // L, group, 0)
            pltpu.sync_copy(gbA, acc.at[ibA.at[1]], add=True)

        pltpu.sync_copy(edg.at[c, gbase], ib0)
        pltpu.async_copy(src_hbm.at[ib0.at[0]], gb0, sem0)

        def pair(p, _):
            step(2 * p, ib0, gb0, sem0, ib1, gb1, sem1)
            step(2 * p + 1, ib1, gb1, sem1, ib0, gb0, sem0)
            return 0
        lax.fori_loop(0, NCHUNKS // 2, pair, 0)
        pltpu.make_async_copy(src_hbm.at[ib0.at[0]], gb0, sem0).wait()

    layer(x0, accB)
    plsc.subcore_barrier()

    def mid(i, _):
        r0 = rbase + i * K
        pltpu.sync_copy(accB.at[pl.ds(r0, K)], gb0)
        pltpu.sync_copy(gb0, e1.at[pl.ds(hbase + r0, K)])
        pltpu.sync_copy(x0.at[pl.ds(hbase + r0, K)], b2)

        def addrow(r, _):
            for j in range(DH // L):
                sl = pl.ds(j * L, L)
                b2[r, sl] = b2[r, sl] + gb0[r, sl]
            return 0
        lax.fori_loop(0, K, addrow, 0)
        pltpu.sync_copy(b2, accA.at[pl.ds(r0, K)])
        return 0
    lax.fori_loop(0, NROWCHUNKS, mid, 0)
    plsc.subcore_barrier()

    layer(e1, accA)
    plsc.subcore_barrier()

    def fin(i, _):
        r0 = rbase + i * K
        pltpu.sync_copy(accA.at[pl.ds(r0, K)], gb0)
        pltpu.sync_copy(gb0, out.at[pl.ds(hbase + r0, K)])
        return 0
    lax.fori_loop(0, NROWCHUNKS, fin, 0)


@jax.jit
def _run(x0, edg):
    mesh = plsc.VectorSubcoreMesh(
        core_axis_name="c", subcore_axis_name="s",
        num_cores=NC, num_subcores=NS)
    kfn = pl.kernel(
        _body,
        out_type=[
            jax.ShapeDtypeStruct((NC * NPAD, DH), jnp.float32),
            jax.ShapeDtypeStruct((NC * NPAD, DH), jnp.float32),
        ],
        mesh=mesh,
        compiler_params=pltpu.CompilerParams(
            use_tc_tiling_on_sc=False, needs_layout_passes=False),
        scratch_types=[
            pltpu.VMEM_SHARED((NPAD, DH), jnp.float32),
            pltpu.VMEM_SHARED((NPAD, DH), jnp.float32),
            pltpu.VMEM((3, K), jnp.int32),
            pltpu.VMEM((3, K), jnp.int32),
            pltpu.VMEM((K, DH), jnp.float32),
            pltpu.VMEM((K, DH), jnp.float32),
            pltpu.VMEM((K, DH), jnp.float32),
            pltpu.SemaphoreType.DMA,
            pltpu.SemaphoreType.DMA,
        ],
    )
    return kfn(x0, edg)


def kernel(adj_indices, adj_values, uEmbeds, iEmbeds):
    emb = jnp.concatenate([uEmbeds, iEmbeds], axis=0)
    embp = jnp.zeros((NPAD, D), jnp.float32).at[:N].set(emb)
    x0 = jnp.concatenate([embp[:, :DH], embp[:, DH:]], axis=0)

    rows = adj_indices[0].astype(jnp.int32)
    cols = adj_indices[1].astype(jnp.int32)
    vals_i = lax.bitcast_convert_type(adj_values.astype(jnp.float32),
                                      jnp.int32)
    pad = ((0, 0), (0, EPT_PAD - EPT))
    rows_p = jnp.pad(rows.reshape(NS, EPT), pad).reshape(NS, NCHUNKS, K)
    cols_p = jnp.pad(cols.reshape(NS, EPT), pad).reshape(NS, NCHUNKS, K)
    vals_p = jnp.pad(vals_i.reshape(NS, EPT), pad).reshape(NS, NCHUNKS, K)

    def pack(col_chunks):
        e = jnp.stack([col_chunks, rows_p, vals_p], axis=2)
        e = e.reshape(NS * NCHUNKS, 3, K)
        return jnp.pad(e, ((0, 1), (0, 0), (0, 0)))
    edg = jnp.stack([pack(cols_p), pack(cols_p + NPAD)])

    out, _e1 = _run(x0, edg)
    total = jnp.concatenate([out[:N], out[NPAD:NPAD + N]], axis=1)
    return total[:N_USER], total[N_USER:]


# speedup vs baseline: 2.9953x; 1.0016x over previous
import jax
import jax.numpy as jnp
from jax import lax
from jax.experimental import pallas as pl
from jax.experimental.pallas import tpu as pltpu
from jax.experimental.pallas import tpu_sc as plsc

N_USER = 5000
N_ITEM = 5000
N = N_USER + N_ITEM
D = 128
DH = D // 2
NC = 2
NS = 16
L = 16
NPAD = 10240
ROWS_PER_TILE = NPAD // NS
K = 128
NROWCHUNKS = ROWS_PER_TILE // K
KO = 512
NSUB = KO // K
E = 320000
EPT = E // NS
NCH = 40
EPT_PAD = NCH * KO
NGCH = NS * NCH + 2


def _body(x0, edg, out, e1, acc,
          ib0, ib1, ib2, ib3, gb0, gb1,
          gsem0, gsem1, ssem0, ssem1, isem0, isem1, isem2, isem3):
    ib = (ib0, ib1, ib2, ib3)
    gb = (gb0, gb1)
    gsem = (gsem0, gsem1)
    ssem = (ssem0, ssem1)
    isem = (isem0, isem1, isem2, isem3)

    c = lax.axis_index("c")
    s = lax.axis_index("s")
    rbase = s * ROWS_PER_TILE
    gbase = s * NCH
    hbase = c * NPAD

    def zrow(r, _):
        for j in range(DH // L):
            gb1[r, pl.ds(j * L, L)] = jnp.zeros((L,), jnp.float32)
        return 0
    lax.fori_loop(0, K, zrow, 0)

    def zchunk(i, _):
        pltpu.sync_copy(gb1.at[pl.ds(0, K)], acc.at[pl.ds(rbase + i * K, K)])
        return 0
    lax.fori_loop(0, NROWCHUNKS, zchunk, 0)
    plsc.subcore_barrier()

    def scale(gbuf, ibuf):
        def group(g, _):
            val16 = plsc.bitcast(
                ibuf[2 * NSUB + g // (K // L), pl.ds((g % (K // L)) * L, L)],
                jnp.float32)
            for el in range(L):
                e = g * L + el
                v = val16[el]
                for j in range(DH // L):
                    sl = pl.ds(j * L, L)
                    gbuf[e, sl] = gbuf[e, sl] * v
            return 0
        lax.fori_loop(0, KO // L, group, 0)

    def layer(src_hbm, acc):
        def start_gathers(t_ph, tb):
            for j in range(NSUB):
                pltpu.async_copy(src_hbm.at[ib[t_ph].at[j]],
                                 gb[tb].at[pl.ds(j * K, K)], gsem[tb])

        def start_scatters(t_ph, tb):
            for j in range(NSUB):
                pltpu.async_copy(gb[tb].at[pl.ds(j * K, K)],
                                 acc.at[ib[t_ph].at[NSUB + j]],
                                 ssem[tb], add=True)

        def wait_gathers(tb):
            pltpu.make_async_copy(src_hbm.at[pl.ds(0, KO)], gb[tb],
                                  gsem[tb]).wait()

        def wait_scatters(tb):
            pltpu.make_async_copy(gb[tb], acc.at[pl.ds(0, KO)],
                                  ssem[tb]).wait()

        def wait_idx(i4):
            pltpu.make_async_copy(edg.at[c, gbase], ib[i4], isem[i4]).wait()

        pltpu.sync_copy(edg.at[c, gbase], ib[0])
        pltpu.sync_copy(edg.at[c, gbase + 1], ib[1])
        start_gathers(0, 0)
        start_gathers(1, 1)
        pltpu.async_copy(edg.at[c, gbase + 2], ib[2], isem[2])
        pltpu.async_copy(edg.at[c, gbase + 3], ib[3], isem[3])
        wait_gathers(0)
        scale(gb[0], ib[0])
        start_scatters(0, 0)

        def step(t, tm):
            b, p = tm % 2, (tm - 1) % 2
            i4, i4p, i4n = tm % 4, (tm - 1) % 4, (tm + 2) % 4
            wait_idx(i4)
            wait_scatters(b)
            start_gathers(i4, b)
            pltpu.async_copy(edg.at[c, gbase + t + 2], ib[i4n], isem[i4n])
            wait_gathers(p)
            scale(gb[p], ib[i4p])
            start_scatters(i4p, p)

        def quad(q, _):
            t0 = 2 + 4 * q
            for k in range(4):
                step(t0 + k, 2 + k)
            return 0
        lax.fori_loop(0, (NCH - 4) // 4, quad, 0)
        step(NCH - 2, NCH - 2)
        step(NCH - 1, NCH - 1)

        wait_gathers((NCH - 1) % 2)
        scale(gb[(NCH - 1) % 2], ib[(NCH - 1) % 4])
        start_scatters((NCH - 1) % 4, (NCH - 1) % 2)
        wait_scatters((NCH - 2) % 2)
        wait_scatters((NCH - 1) % 2)
        wait_idx(NCH % 4)
        wait_idx((NCH + 1) % 4)

    layer(x0, acc)
    plsc.subcore_barrier()

    def mid(i, _):
        r0 = rbase + i * K
        pltpu.sync_copy(acc.at[pl.ds(r0, K)], gb0.at[pl.ds(0, K)])
        pltpu.sync_copy(gb0.at[pl.ds(0, K)], e1.at[pl.ds(hbase + r0, K)])
        pltpu.sync_copy(x0.at[pl.ds(hbase + r0, K)], gb1.at[pl.ds(0, K)])

        def addrow(r, _):
            for j in range(DH // L):
                sl = pl.ds(j * L, L)
                gb1[r, sl] = gb1[r, sl] + gb0[r, sl]
            return 0
        lax.fori_loop(0, K, addrow, 0)
        pltpu.sync_copy(gb1.at[pl.ds(0, K)], acc.at[pl.ds(r0, K)])
        return 0
    lax.fori_loop(0, NROWCHUNKS, mid, 0)
    plsc.subcore_barrier()

    layer(e1, acc)
    plsc.subcore_barrier()

    def fin(i, _):
        r0 = rbase + i * K
        pltpu.sync_copy(acc.at[pl.ds(r0, K)], gb0.at[pl.ds(0, K)])
        pltpu.sync_copy(gb0.at[pl.ds(0, K)], out.at[pl.ds(hbase + r0, K)])
        return 0
    lax.fori_loop(0, NROWCHUNKS, fin, 0)


@jax.jit
def _run(x0, edg):
    mesh = plsc.VectorSubcoreMesh(
        core_axis_name="c", subcore_axis_name="s",
        num_cores=NC, num_subcores=NS)
    kfn = pl.kernel(
        _body,
        out_type=[
            jax.ShapeDtypeStruct((NC * NPAD, DH), jnp.float32),
            jax.ShapeDtypeStruct((NC * NPAD, DH), jnp.float32),
        ],
        mesh=mesh,
        compiler_params=pltpu.CompilerParams(
            use_tc_tiling_on_sc=False, needs_layout_passes=False),
        scratch_types=[
            pltpu.VMEM_SHARED((NPAD, DH), jnp.float32),
            pltpu.VMEM((3 * NSUB, K), jnp.int32),
            pltpu.VMEM((3 * NSUB, K), jnp.int32),
            pltpu.VMEM((3 * NSUB, K), jnp.int32),
            pltpu.VMEM((3 * NSUB, K), jnp.int32),
            pltpu.VMEM((KO, DH), jnp.float32),
            pltpu.VMEM((KO, DH), jnp.float32),
            pltpu.SemaphoreType.DMA,
            pltpu.SemaphoreType.DMA,
            pltpu.SemaphoreType.DMA,
            pltpu.SemaphoreType.DMA,
            pltpu.SemaphoreType.DMA,
            pltpu.SemaphoreType.DMA,
            pltpu.SemaphoreType.DMA,
            pltpu.SemaphoreType.DMA,
        ],
    )
    return kfn(x0, edg)


def kernel(adj_indices, adj_values, uEmbeds, iEmbeds):
    emb = jnp.concatenate([uEmbeds, iEmbeds], axis=0)
    embp = jnp.zeros((NPAD, D), jnp.float32).at[:N].set(emb)
    x0 = jnp.concatenate([embp[:, :DH], embp[:, DH:]], axis=0)

    rows = adj_indices[0].astype(jnp.int32)
    cols = adj_indices[1].astype(jnp.int32)
    vals_i = lax.bitcast_convert_type(adj_values.astype(jnp.float32),
                                      jnp.int32)
    pad = ((0, 0), (0, EPT_PAD - EPT))
    shp = (NS, NCH, NSUB, K)
    rows_p = jnp.pad(rows.reshape(NS, EPT), pad).reshape(shp)
    cols_p = jnp.pad(cols.reshape(NS, EPT), pad).reshape(shp)
    vals_p = jnp.pad(vals_i.reshape(NS, EPT), pad).reshape(shp)

    def pack(col_chunks):
        e = jnp.concatenate([col_chunks, rows_p, vals_p], axis=2)
        e = e.reshape(NS * NCH, 3 * NSUB, K)
        return jnp.pad(e, ((0, 2), (0, 0), (0, 0)))
    edg = jnp.stack([pack(cols_p), pack(cols_p + NPAD)])

    out, _e1 = _run(x0, edg)
    total = jnp.concatenate([out[:N], out[NPAD:NPAD + N]], axis=1)
    return total[:N_USER], total[N_USER:]


# speedup vs baseline: 5.3604x; 1.7896x over previous
import jax
import jax.numpy as jnp
from jax import lax
from jax.experimental import pallas as pl
from jax.experimental.pallas import tpu as pltpu
from jax.experimental.pallas import tpu_sc as plsc

N_USER = 5000
N_ITEM = 5000
N = N_USER + N_ITEM
D = 128
DH = D // 2
NC = 2
NS = 16
L = 16
NPAD = 10240
ROWS_PER_TILE = NPAD // NS
K = 128
NROWCHUNKS = ROWS_PER_TILE // K
KO = 512
NSUB = KO // K
E = 320000
EPT = E // NS
NCH = 40
EPT_PAD = NCH * KO
NGCH = NS * NCH + 2


def _body(x0, edg, out, e1, acc,
          ib0, ib1, ib2, ib3, gb0, gb1,
          gsem0, gsem1, ssem0, ssem1, isem0, isem1, isem2, isem3):
    ib = (ib0, ib1, ib2, ib3)
    gb = (gb0, gb1)
    gsem = (gsem0, gsem1)
    ssem = (ssem0, ssem1)
    isem = (isem0, isem1, isem2, isem3)

    c = lax.axis_index("c")
    s = lax.axis_index("s")
    rbase = s * ROWS_PER_TILE
    gbase = s * NCH
    hbase = c * NPAD

    def zrow(r, _):
        for j in range(DH // L):
            gb1[r, pl.ds(j * L, L)] = jnp.zeros((L,), jnp.float32)
        return 0
    lax.fori_loop(0, K, zrow, 0)

    def zchunk(i, _):
        pltpu.sync_copy(gb1.at[pl.ds(0, K)], acc.at[pl.ds(rbase + i * K, K)])
        return 0
    lax.fori_loop(0, NROWCHUNKS, zchunk, 0)
    plsc.subcore_barrier()

    def scale(gbuf, ibuf):
        @plsc.parallel_loop(0, KO // L, unroll=2)
        def group(g):
            val16 = plsc.bitcast(
                ibuf[2 * NSUB + g // (K // L), pl.ds((g % (K // L)) * L, L)],
                jnp.float32)
            for el in range(L):
                e = g * L + el
                v = val16[el]
                for j in range(DH // L):
                    sl = pl.ds(j * L, L)
                    gbuf[e, sl] = gbuf[e, sl] * v

    def layer(src_hbm, acc):
        def start_gathers(t_ph, tb):
            for j in range(NSUB):
                pltpu.async_copy(src_hbm.at[ib[t_ph].at[j]],
                                 gb[tb].at[pl.ds(j * K, K)], gsem[tb])

        def start_scatters(t_ph, tb):
            for j in range(NSUB):
                pltpu.async_copy(gb[tb].at[pl.ds(j * K, K)],
                                 acc.at[ib[t_ph].at[NSUB + j]],
                                 ssem[tb], add=True)

        def wait_gathers(tb):
            pltpu.make_async_copy(src_hbm.at[pl.ds(0, KO)], gb[tb],
                                  gsem[tb]).wait()

        def wait_scatters(tb):
            pltpu.make_async_copy(gb[tb], acc.at[pl.ds(0, KO)],
                                  ssem[tb]).wait()

        def wait_idx(i4):
            pltpu.make_async_copy(edg.at[c, gbase], ib[i4], isem[i4]).wait()

        pltpu.sync_copy(edg.at[c, gbase], ib[0])
        pltpu.sync_copy(edg.at[c, gbase + 1], ib[1])
        start_gathers(0, 0)
        start_gathers(1, 1)
        pltpu.async_copy(edg.at[c, gbase + 2], ib[2], isem[2])
        pltpu.async_copy(edg.at[c, gbase + 3], ib[3], isem[3])
        wait_gathers(0)
        scale(gb[0], ib[0])
        start_scatters(0, 0)

        def step(t, tm):
            b, p = tm % 2, (tm - 1) % 2
            i4, i4p, i4n = tm % 4, (tm - 1) % 4, (tm + 2) % 4
            wait_idx(i4)
            wait_scatters(b)
            start_gathers(i4, b)
            pltpu.async_copy(edg.at[c, gbase + t + 2], ib[i4n], isem[i4n])
            wait_gathers(p)
            scale(gb[p], ib[i4p])
            start_scatters(i4p, p)

        def quad(q, _):
            t0 = 2 + 4 * q
            for k in range(4):
                step(t0 + k, 2 + k)
            return 0
        lax.fori_loop(0, (NCH - 4) // 4, quad, 0)
        step(NCH - 2, NCH - 2)
        step(NCH - 1, NCH - 1)

        wait_gathers((NCH - 1) % 2)
        scale(gb[(NCH - 1) % 2], ib[(NCH - 1) % 4])
        start_scatters((NCH - 1) % 4, (NCH - 1) % 2)
        wait_scatters((NCH - 2) % 2)
        wait_scatters((NCH - 1) % 2)
        wait_idx(NCH % 4)
        wait_idx((NCH + 1) % 4)

    layer(x0, acc)
    plsc.subcore_barrier()

    def mid(i, _):
        r0 = rbase + i * K
        pltpu.sync_copy(acc.at[pl.ds(r0, K)], gb0.at[pl.ds(0, K)])
        pltpu.sync_copy(gb0.at[pl.ds(0, K)], e1.at[pl.ds(hbase + r0, K)])
        pltpu.sync_copy(x0.at[pl.ds(hbase + r0, K)], gb1.at[pl.ds(0, K)])

        def addrow(r, _):
            for j in range(DH // L):
                sl = pl.ds(j * L, L)
                gb1[r, sl] = gb1[r, sl] + gb0[r, sl]
            return 0
        lax.fori_loop(0, K, addrow, 0)
        pltpu.sync_copy(gb1.at[pl.ds(0, K)], acc.at[pl.ds(r0, K)])
        return 0
    lax.fori_loop(0, NROWCHUNKS, mid, 0)
    plsc.subcore_barrier()

    layer(e1, acc)
    plsc.subcore_barrier()

    def fin(i, _):
        r0 = rbase + i * K
        pltpu.sync_copy(acc.at[pl.ds(r0, K)], gb0.at[pl.ds(0, K)])
        pltpu.sync_copy(gb0.at[pl.ds(0, K)], out.at[pl.ds(hbase + r0, K)])
        return 0
    lax.fori_loop(0, NROWCHUNKS, fin, 0)


@jax.jit
def _run(x0, edg):
    mesh = plsc.VectorSubcoreMesh(
        core_axis_name="c", subcore_axis_name="s",
        num_cores=NC, num_subcores=NS)
    kfn = pl.kernel(
        _body,
        out_type=[
            jax.ShapeDtypeStruct((NC * NPAD, DH), jnp.float32),
            jax.ShapeDtypeStruct((NC * NPAD, DH), jnp.float32),
        ],
        mesh=mesh,
        compiler_params=pltpu.CompilerParams(
            use_tc_tiling_on_sc=False, needs_layout_passes=False),
        scratch_types=[
            pltpu.VMEM_SHARED((NPAD, DH), jnp.float32),
            pltpu.VMEM((3 * NSUB, K), jnp.int32),
            pltpu.VMEM((3 * NSUB, K), jnp.int32),
            pltpu.VMEM((3 * NSUB, K), jnp.int32),
            pltpu.VMEM((3 * NSUB, K), jnp.int32),
            pltpu.VMEM((KO, DH), jnp.float32),
            pltpu.VMEM((KO, DH), jnp.float32),
            pltpu.SemaphoreType.DMA,
            pltpu.SemaphoreType.DMA,
            pltpu.SemaphoreType.DMA,
            pltpu.SemaphoreType.DMA,
            pltpu.SemaphoreType.DMA,
            pltpu.SemaphoreType.DMA,
            pltpu.SemaphoreType.DMA,
            pltpu.SemaphoreType.DMA,
        ],
    )
    return kfn(x0, edg)


def kernel(adj_indices, adj_values, uEmbeds, iEmbeds):
    emb = jnp.concatenate([uEmbeds, iEmbeds], axis=0)
    embp = jnp.zeros((NPAD, D), jnp.float32).at[:N].set(emb)
    x0 = jnp.concatenate([embp[:, :DH], embp[:, DH:]], axis=0)

    rows = adj_indices[0].astype(jnp.int32)
    cols = adj_indices[1].astype(jnp.int32)
    vals_i = lax.bitcast_convert_type(adj_values.astype(jnp.float32),
                                      jnp.int32)
    pad = ((0, 0), (0, EPT_PAD - EPT))
    shp = (NS, NCH, NSUB, K)
    rows_p = jnp.pad(rows.reshape(NS, EPT), pad).reshape(shp)
    cols_p = jnp.pad(cols.reshape(NS, EPT), pad).reshape(shp)
    vals_p = jnp.pad(vals_i.reshape(NS, EPT), pad).reshape(shp)

    def pack(col_chunks):
        e = jnp.concatenate([col_chunks, rows_p, vals_p], axis=2)
        e = e.reshape(NS * NCH, 3 * NSUB, K)
        return jnp.pad(e, ((0, 2), (0, 0), (0, 0)))
    edg = jnp.stack([pack(cols_p), pack(cols_p + NPAD)])

    out, _e1 = _run(x0, edg)
    total = jnp.concatenate([out[:N], out[NPAD:NPAD + N]], axis=1)
    return total[:N_USER], total[N_USER:]
